# Initial kernel scaffold; baseline (speedup 1.0000x reference)
import jax
import jax.numpy as jnp
from jax.experimental import pallas as pl


def kernel(x, conv1_w, conv1_b, layer1_w, layer1_b, stage0_w_s2, stage0_b_s2, stage0_w_ds, stage0_b_ds, stage0_w_in, stage0_b_in, stage1_w_s2, stage1_b_s2, stage1_w_ds, stage1_b_ds, stage1_w_in, stage1_b_in, stage2_w_s2, stage2_b_s2, stage2_w_ds, stage2_b_ds, stage2_w_in, stage2_b_in, centers, c2, thres):
    raise NotImplementedError("write your pallas kernel here")


# speedup vs baseline: 1.0002x; 1.0002x over previous
import reference as _r


def kernel(*args):
    return _r.reference(*args)


# speedup vs baseline: 18.1977x; 18.1944x over previous
import functools

import numpy as np
import jax
import jax.numpy as jnp
from jax.experimental import pallas as pl
from jax.experimental.pallas import tpu as pltpu

_PAR = pltpu.CompilerParams(dimension_semantics=("parallel",))


def _ru(v, m):
    return (v + m - 1) // m * m


def _nlen(h, w):
    return _ru((h + 4) * (w + 2), 128)


_N56, _WS56 = _nlen(56, 56), 58
_N28, _WS28 = _nlen(28, 28), 30
_N14, _WS14 = _nlen(14, 14), 16
_N7, _WS7 = _nlen(7, 7), 9


def _rot(x, st, n):
    return pltpu.roll(x, (-st) % n, 1) if st else x


def _vmask(h, w, n):
    ws = w + 2
    i = jax.lax.broadcasted_iota(jnp.int32, (1, n), 1)
    y = i // ws - 1
    x = i % ws - 1
    ok = (y >= 0) & (y < h) & (x >= 0) & (x < w)
    return ok.astype(jnp.float32)


def _conv3(xv, w, b, mask, ws, n, residual=None):
    taps = [_rot(xv, dy * ws + dx, n) for dy in (-1, 0, 1) for dx in (-1, 0, 1)]
    slab = jnp.concatenate(taps, axis=0).astype(jnp.bfloat16)
    y = jnp.dot(w, slab, preferred_element_type=jnp.float32) + b
    if residual is not None:
        y = y + residual
    return jnp.maximum(y, 0.0) * mask


def _sel_matrix(h, w):
    wsi, ni = w + 2, _nlen(h, w)
    ho, wo = h // 2, w // 2
    wso, no = wo + 2, _nlen(ho, wo)
    s = np.zeros((ni, 4 * no), np.float32)
    for a in (0, 1):
        for b in (0, 1):
            for y2 in range(ho):
                for x2 in range(wo):
                    src = (2 * y2 + a + 1) * wsi + (2 * x2 + b + 1)
                    dst = (a * 2 + b) * no + (y2 + 1) * wso + (x2 + 1)
                    s[src, dst] = 1.0
    return jnp.asarray(s, jnp.bfloat16)


def _front_kernel(xph_ref, w1_ref, b1_ref, l1w_ref, l1b_ref, s1_ref, o_ref):
    n, ws = _N56, _WS56
    m56 = _vmask(56, 56, n)
    xin = xph_ref[0].astype(jnp.float32)
    phs = [xin[3 * p:3 * p + 3] for p in range(16)]

    rolled = {}

    def tap(py, px, sy, sx):
        key = (py, px, sy, sx)
        if key not in rolled:
            rolled[key] = _rot(phs[py * 4 + px], sy * ws + sx, n)
        return rolled[key]

    souts = []
    for a in (0, 1):
        for b in (0, 1):
            pieces = []
            for u in range(7):
                oy = 2 * a + u - 3
                py, sy = oy % 4, oy // 4
                for v in range(7):
                    ox = 2 * b + v - 3
                    px, sx = ox % 4, ox // 4
                    pieces.append(tap(py, px, sy, sx))
            slab = jnp.concatenate(pieces, axis=0).astype(jnp.bfloat16)
            y = jnp.dot(w1_ref[...], slab, preferred_element_type=jnp.float32)
            souts.append(jnp.maximum(y + b1_ref[...], 0.0) * m56)

    cur = None
    for oi in (-1, 0, 1):
        for oj in (-1, 0, 1):
            t = _rot(souts[(oi % 2) * 2 + (oj % 2)], (oi // 2) * ws + (oj // 2), n)
            cur = t if cur is None else jnp.maximum(cur, t)
    x = cur * m56

    for blk in range(2):
        y = _conv3(x, l1w_ref[2 * blk], l1b_ref[2 * blk], m56, ws, n)
        x = _conv3(y, l1w_ref[2 * blk + 1], l1b_ref[2 * blk + 1], m56, ws, n,
                   residual=x)

    ph = jnp.dot(x.astype(jnp.bfloat16), s1_ref[...],
                 preferred_element_type=jnp.float32)
    for k in range(4):
        o_ref[0, k] = ph[:, k * _N28:(k + 1) * _N28].astype(o_ref.dtype)


def _call_front(xph, w1, b1, l1w, l1b, s1):
    bsz = xph.shape[0]
    full = lambda a: pl.BlockSpec(a.shape, lambda i: (0,) * a.ndim)
    return pl.pallas_call(
        _front_kernel,
        out_shape=jax.ShapeDtypeStruct((bsz, 4, 64, _N28), jnp.bfloat16),
        grid=(bsz,),
        in_specs=[
            pl.BlockSpec((1,) + xph.shape[1:], lambda i: (i, 0, 0)),
            full(w1), full(b1), full(l1w), full(l1b), full(s1),
        ],
        out_specs=pl.BlockSpec((1, 4, 64, _N28), lambda i: (i, 0, 0, 0)),
        compiler_params=_PAR,
    )(xph, w1, b1, l1w, l1b, s1)


def _stage_body(p, ws2, bs2, wds, bds, wi, bi, mask, ws, n):
    pieces = []
    for i in range(3):
        oy = i - 1
        pa, sy = oy % 2, oy // 2
        for j in range(3):
            ox = j - 1
            pb, sx = ox % 2, ox // 2
            pieces.append(_rot(p[pa * 2 + pb], sy * ws + sx, n))
    slab = jnp.concatenate(pieces, axis=0).astype(jnp.bfloat16)
    y1 = jnp.dot(ws2, slab, preferred_element_type=jnp.float32) + bs2
    y1 = jnp.maximum(y1, 0.0) * mask
    idn = (jnp.dot(wds, p[0].astype(jnp.bfloat16),
                   preferred_element_type=jnp.float32) + bds) * mask
    x = _conv3(y1, wi[0], bi[0], mask, ws, n, residual=idn)
    y2 = _conv3(x, wi[1], bi[1], mask, ws, n)
    return _conv3(y2, wi[2], bi[2], mask, ws, n, residual=x)


def _split4(x, s_ref, nout):
    ph = jnp.dot(x.astype(jnp.bfloat16), s_ref[...],
                 preferred_element_type=jnp.float32)
    return [ph[:, k * nout:(k + 1) * nout] for k in range(4)]


def _back_kernel(ph_ref,
                 w02, b02, wd0, bd0, wi0, bi0,
                 w12, b12, wd1, bd1, wi1, bi1,
                 w22, b22, wd2, bd2, wi2, bi2,
                 s2_ref, s3_ref, cen_ref, c2_ref, thr_ref, o_ref):
    m28 = _vmask(28, 28, _N28)
    m14 = _vmask(14, 14, _N14)
    m7 = _vmask(7, 7, _N7)

    p = [ph_ref[0, k].astype(jnp.float32) for k in range(4)]
    x = _stage_body(p, w02[...], b02[...], wd0[...], bd0[...], wi0, bi0,
                    m28, _WS28, _N28)
    p = _split4(x, s2_ref, _N14)
    x = _stage_body(p, w12[...], b12[...], wd1[...], bd1[...], wi1, bi1,
                    m14, _WS14, _N14)
    p = _split4(x, s3_ref, _N7)
    x = _stage_body(p, w22[...], b22[...], wd2[...], bd2[...], wi2, bi2,
                    m7, _WS7, _N7)

    inv_area = 1.0 / 49.0
    fcol = jnp.sum(x, axis=1, keepdims=True) * inv_area
    f2 = jnp.sum(fcol * fcol, axis=0, keepdims=True)
    pxm = jnp.dot(cen_ref[...], x, preferred_element_type=jnp.float32)
    fc = jnp.sum(pxm, axis=1, keepdims=True) * inv_area
    dist = f2 - 2.0 * fc + c2_ref[...]
    o_ref[0] = -(dist - thr_ref[...]) * 0.5


def _call_back(ph, sp, s2, s3, cen, c2, thr):
    bsz = ph.shape[0]
    cls = cen.shape[0]
    args = list(sp) + [s2, s3, cen, c2, thr]
    full = lambda a: pl.BlockSpec(a.shape, lambda i: (0,) * a.ndim)
    return pl.pallas_call(
        _back_kernel,
        out_shape=jax.ShapeDtypeStruct((bsz, cls, 1), jnp.float32),
        grid=(bsz,),
        in_specs=[pl.BlockSpec((1,) + ph.shape[1:], lambda i: (i, 0, 0, 0))]
                 + [full(a) for a in args],
        out_specs=pl.BlockSpec((1, cls, 1), lambda i: (i, 0, 0)),
        compiler_params=_PAR,
    )(ph, *args)


def kernel(x, conv1_w, conv1_b, layer1_w, layer1_b,
           stage0_w_s2, stage0_b_s2, stage0_w_ds, stage0_b_ds,
           stage0_w_in, stage0_b_in,
           stage1_w_s2, stage1_b_s2, stage1_w_ds, stage1_b_ds,
           stage1_w_in, stage1_b_in,
           stage2_w_s2, stage2_b_s2, stage2_w_ds, stage2_b_ds,
           stage2_w_in, stage2_b_in,
           centers, c2, thres):
    bsz = x.shape[0]
    t = x.astype(jnp.bfloat16).reshape(bsz, 3, 56, 4, 56, 4)
    t = t.transpose(0, 3, 5, 1, 2, 4).reshape(bsz, 48, 56, 56)
    t = jnp.pad(t, ((0, 0), (0, 0), (1, 0), (1, 1))).reshape(bsz, 48, 57 * 58)
    xph = jnp.pad(t, ((0, 0), (0, 0), (0, _N56 - 57 * 58)))

    s1 = _sel_matrix(56, 56)
    s2 = _sel_matrix(28, 28)
    s3 = _sel_matrix(14, 14)

    ph = _call_front(xph, conv1_w, conv1_b, layer1_w, layer1_b, s1)
    sp = (stage0_w_s2, stage0_b_s2, stage0_w_ds, stage0_b_ds,
          stage0_w_in, stage0_b_in,
          stage1_w_s2, stage1_b_s2, stage1_w_ds, stage1_b_ds,
          stage1_w_in, stage1_b_in,
          stage2_w_s2, stage2_b_s2, stage2_w_ds, stage2_b_ds,
          stage2_w_in, stage2_b_in)
    logits = _call_back(ph, sp, s2, s3, centers, c2, thres)
    return logits[:, :, 0]
